# Initial kernel scaffold; baseline (speedup 1.0000x reference)
import jax
import jax.numpy as jnp
from jax.experimental import pallas as pl


def kernel(x_user, x_item, edge_index_ui, edge_index_iu, Wp, bp, W_ui0, as_ui0, ad_ui0, b_ui0, W_iu0, as_iu0, ad_iu0, b_iu0, W_ui1, as_ui1, ad_ui1, b_ui1, W_iu1, as_iu1, ad_iu1, b_iu1, Wo, bo):
    raise NotImplementedError("write your pallas kernel here")


# speedup vs baseline: 3.9965x; 3.9965x over previous
import functools

import jax
import jax.numpy as jnp
from jax import lax
from jax.experimental import pallas as pl
from jax.experimental.pallas import tpu as pltpu
from jax.experimental.pallas import tpu_sc as plsc

N = 10000
E = 160000
D = 128
OUT = 64
NC = 2
NS = 16
NW = NC * NS
L = 16
RPT = D // NW
CHUNK = 320
SLOTS = 2048

_f32 = jnp.float32


def _one_gat(hs_hbm, als_hbm, ald_hbm, src_hbm, dst_hbm, num_hbm, den_hbm,
             table_v, acc_v, als_v, ald_v, den_v, slots_v, src_b, dst_b, wid):
    base = wid * (RPT * N)
    pltpu.sync_copy(hs_hbm.at[pl.ds(base, RPT * N)], table_v)
    pltpu.sync_copy(als_hbm, als_v)
    pltpu.sync_copy(ald_hbm, ald_v)

    zeros16 = jnp.zeros((L,), _f32)

    def _zacc(i, c):
        acc_v[pl.ds(i * L, L)] = zeros16
        return c
    lax.fori_loop(0, (RPT * N) // L, _zacc, 0)

    def _zden(i, c):
        den_v[pl.ds(i * L, L)] = zeros16
        return c
    lax.fori_loop(0, N // L, _zden, 0)

    iota16 = lax.iota(jnp.int32, L)

    def _chunk(ci, c):
        off = ci * CHUNK
        pltpu.sync_copy(src_hbm.at[pl.ds(off, CHUNK)], src_b)
        pltpu.sync_copy(dst_hbm.at[pl.ds(off, CHUNK)], dst_b)
        for j in range(CHUNK // L):
            sv = src_b[pl.ds(j * L, L)]
            dv = dst_b[pl.ds(j * L, L)]
            a = plsc.load_gather(als_v, [sv]) + plsc.load_gather(ald_v, [dv])
            a = jnp.maximum(a, a * 0.2)
            ex = jnp.exp(a)
            vals = [plsc.load_gather(table_v, [sv + (r * N)]) * ex
                    for r in range(RPT)]
            slot = jnp.bitwise_and(dv, SLOTS - 1)

            def _wcond(carry):
                it, rem = carry
                return jnp.logical_and(it < L, jnp.max(rem) > 0)

            def _wbody(carry, dv=dv, slot=slot, vals=vals, ex=ex):
                it, rem = carry
                remb = rem > 0
                plsc.store_scatter(slots_v, [slot], iota16, mask=remb)
                back = plsc.load_gather(slots_v, [slot])
                win = jnp.logical_and(back == iota16, remb)
                for r in range(RPT):
                    plsc.addupdate_scatter(acc_v, [dv + (r * N)], vals[r],
                                           mask=win)
                plsc.addupdate_scatter(den_v, [dv], ex, mask=win)
                return (it + jnp.int32(1), jnp.where(win, 0, rem))

            lax.while_loop(_wcond, _wbody,
                           (jnp.int32(0), jnp.ones((L,), jnp.int32)))
        return c
    lax.fori_loop(0, E // CHUNK, _chunk, 0)

    pltpu.sync_copy(acc_v, num_hbm.at[pl.ds(base, RPT * N)])

    @pl.when(wid == 0)
    def _():
        pltpu.sync_copy(den_v, den_hbm)


def _sc_layer_body(hsA_hbm, alsA_hbm, aldA_hbm, srcA_hbm, dstA_hbm,
                   hsB_hbm, alsB_hbm, aldB_hbm, srcB_hbm, dstB_hbm,
                   numA_hbm, denA_hbm, numB_hbm, denB_hbm,
                   table_v, acc_v, als_v, ald_v, den_v, slots_v, src_b, dst_b):
    wid = lax.axis_index("s") * NC + lax.axis_index("c")
    _one_gat(hsA_hbm, alsA_hbm, aldA_hbm, srcA_hbm, dstA_hbm,
             numA_hbm, denA_hbm,
             table_v, acc_v, als_v, ald_v, den_v, slots_v, src_b, dst_b, wid)
    _one_gat(hsB_hbm, alsB_hbm, aldB_hbm, srcB_hbm, dstB_hbm,
             numB_hbm, denB_hbm,
             table_v, acc_v, als_v, ald_v, den_v, slots_v, src_b, dst_b, wid)


_sc_layer = pl.kernel(
    _sc_layer_body,
    out_type=(
        jax.ShapeDtypeStruct((D * N,), _f32),
        jax.ShapeDtypeStruct((N,), _f32),
        jax.ShapeDtypeStruct((D * N,), _f32),
        jax.ShapeDtypeStruct((N,), _f32),
    ),
    mesh=plsc.VectorSubcoreMesh(core_axis_name="c", subcore_axis_name="s"),
    compiler_params=pltpu.CompilerParams(needs_layout_passes=False),
    scratch_types=[
        pltpu.VMEM((RPT * N,), _f32),
        pltpu.VMEM((RPT * N,), _f32),
        pltpu.VMEM((N,), _f32),
        pltpu.VMEM((N,), _f32),
        pltpu.VMEM((N,), _f32),
        pltpu.VMEM((SLOTS,), jnp.int32),
        pltpu.VMEM((CHUNK,), jnp.int32),
        pltpu.VMEM((CHUNK,), jnp.int32),
    ],
)


_DN_T = (((0,), (1,)), ((), ()))
_DN_00 = (((0,), (0,)), ((), ()))
_DN_01 = (((0,), (1,)), ((), ()))


def _elu(x):
    return jnp.where(x > 0, x, jnp.exp(x) - 1.0)


def _gat_prep(hsrcT, hdstT, w, a_s, a_d, hs_o, als_o, ald_o):
    hs = lax.dot_general(w, hsrcT, _DN_00, preferred_element_type=_f32)
    hs_o[...] = hs
    als_o[...] = lax.dot_general(hs, a_s, _DN_01, preferred_element_type=_f32)
    wd = lax.dot_general(w, a_d, (((1,), (1,)), ((), ())),
                         preferred_element_type=_f32)
    ald_o[...] = lax.dot_general(hdstT, wd, _DN_00,
                                 preferred_element_type=_f32)


def _tc_prep_body(xu, xi, wp, bp, wA, asA, adA, wB, asB, adB,
                  hsA_o, alsA_o, aldA_o, hsB_o, alsB_o, aldB_o):
    huT = lax.dot_general(wp[...], xu[...], _DN_T,
                          preferred_element_type=_f32) + bp[...]
    hiT = lax.dot_general(wp[...], xi[...], _DN_T,
                          preferred_element_type=_f32) + bp[...]
    _gat_prep(hiT, huT, wA[...], asA[...], adA[...], hsA_o, alsA_o, aldA_o)
    _gat_prep(huT, hiT, wB[...], asB[...], adB[...], hsB_o, alsB_o, aldB_o)


_tc_prep = pl.pallas_call(
    _tc_prep_body,
    out_shape=[
        jax.ShapeDtypeStruct((D, N), _f32),
        jax.ShapeDtypeStruct((N, 1), _f32),
        jax.ShapeDtypeStruct((N, 1), _f32),
        jax.ShapeDtypeStruct((D, N), _f32),
        jax.ShapeDtypeStruct((N, 1), _f32),
        jax.ShapeDtypeStruct((N, 1), _f32),
    ],
)


def _tc_layer_body(numA, denA, bA, numB, denB, bB, wA, asA, adA, wB, asB, adB,
                   hsA_o, alsA_o, aldA_o, hsB_o, alsB_o, aldB_o):
    huT = _elu(numA[...] / (denA[...] + 1e-16) + bA[...])
    hiT = _elu(numB[...] / (denB[...] + 1e-16) + bB[...])
    _gat_prep(hiT, huT, wA[...], asA[...], adA[...], hsA_o, alsA_o, aldA_o)
    _gat_prep(huT, hiT, wB[...], asB[...], adB[...], hsB_o, alsB_o, aldB_o)


_tc_layer = pl.pallas_call(
    _tc_layer_body,
    out_shape=[
        jax.ShapeDtypeStruct((D, N), _f32),
        jax.ShapeDtypeStruct((N, 1), _f32),
        jax.ShapeDtypeStruct((N, 1), _f32),
        jax.ShapeDtypeStruct((D, N), _f32),
        jax.ShapeDtypeStruct((N, 1), _f32),
        jax.ShapeDtypeStruct((N, 1), _f32),
    ],
)


def _tc_final_body(numA, denA, bA, numB, denB, bB, wo, bo, outu_o, hiT_o):
    hu2T = _elu(numA[...] / (denA[...] + 1e-16) + bA[...])
    hiT_o[...] = _elu(numB[...] / (denB[...] + 1e-16) + bB[...])
    outu_o[...] = lax.dot_general(hu2T, wo[...], _DN_00,
                                  preferred_element_type=_f32) + bo[...]


_tc_final = pl.pallas_call(
    _tc_final_body,
    out_shape=[
        jax.ShapeDtypeStruct((N, OUT), _f32),
        jax.ShapeDtypeStruct((D, N), _f32),
    ],
)


def kernel(x_user, x_item, edge_index_ui, edge_index_iu, Wp, bp,
           W_ui0, as_ui0, ad_ui0, b_ui0, W_iu0, as_iu0, ad_iu0, b_iu0,
           W_ui1, as_ui1, ad_ui1, b_ui1, W_iu1, as_iu1, ad_iu1, b_iu1,
           Wo, bo):
    srcA = edge_index_iu[0].astype(jnp.int32)
    dstA = edge_index_iu[1].astype(jnp.int32)
    srcB = edge_index_ui[0].astype(jnp.int32)
    dstB = edge_index_ui[1].astype(jnp.int32)

    hsA, alsA, aldA, hsB, alsB, aldB = _tc_prep(
        x_user, x_item, Wp, bp.reshape(D, 1),
        W_iu0, as_iu0, ad_iu0, W_ui0, as_ui0, ad_ui0)

    numA, denA, numB, denB = _sc_layer(
        hsA.reshape(-1), alsA.reshape(-1), aldA.reshape(-1), srcA, dstA,
        hsB.reshape(-1), alsB.reshape(-1), aldB.reshape(-1), srcB, dstB)

    hsA1, alsA1, aldA1, hsB1, alsB1, aldB1 = _tc_layer(
        numA.reshape(D, N), denA.reshape(1, N), b_iu0.reshape(D, 1),
        numB.reshape(D, N), denB.reshape(1, N), b_ui0.reshape(D, 1),
        W_iu1, as_iu1, ad_iu1, W_ui1, as_ui1, ad_ui1)

    numA1, denA1, numB1, denB1 = _sc_layer(
        hsA1.reshape(-1), alsA1.reshape(-1), aldA1.reshape(-1), srcA, dstA,
        hsB1.reshape(-1), alsB1.reshape(-1), aldB1.reshape(-1), srcB, dstB)

    out_user, hi2T = _tc_final(
        numA1.reshape(D, N), denA1.reshape(1, N), b_iu1.reshape(D, 1),
        numB1.reshape(D, N), denB1.reshape(1, N), b_ui1.reshape(D, 1),
        Wo, bo.reshape(1, OUT))

    return (out_user, jnp.transpose(hi2T))


# speedup vs baseline: 11.6429x; 2.9133x over previous
import functools

import jax
import jax.numpy as jnp
from jax import lax
from jax.experimental import pallas as pl
from jax.experimental.pallas import tpu as pltpu
from jax.experimental.pallas import tpu_sc as plsc

N = 10000
E = 160000
D = 128
OUT = 64
NC = 2
NS = 16
NW = NC * NS
L = 16
RPT = D // NW
CHUNK = 320
SLOTS = 2048

_f32 = jnp.float32


def _edge_group(sv, dv, table_v, acc_v, als_v, ald_v, den_v, slots_v, iota16,
                rem):
    a = plsc.load_gather(als_v, [sv]) + plsc.load_gather(ald_v, [dv])
    a = jnp.maximum(a, a * 0.2)
    ex = jnp.exp(a)
    vals = [plsc.load_gather(table_v, [sv + (r * N)]) * ex
            for r in range(RPT)]
    slot = jnp.bitwise_and(dv, SLOTS - 1)
    plsc.store_scatter(slots_v, [slot], iota16, mask=rem)
    back = plsc.load_gather(slots_v, [slot])
    win = jnp.logical_and(back == iota16, rem)
    for r in range(RPT):
        plsc.addupdate_scatter(acc_v, [dv + (r * N)], vals[r], mask=win)
    plsc.addupdate_scatter(den_v, [dv], ex, mask=win)
    return jnp.logical_and(rem, jnp.logical_not(win))


def _resolve_group(sv, dv, table_v, acc_v, als_v, ald_v, den_v, slots_v,
                   iota16, valid):
    a = plsc.load_gather(als_v, [sv]) + plsc.load_gather(ald_v, [dv])
    a = jnp.maximum(a, a * 0.2)
    ex = jnp.exp(a)
    vals = [plsc.load_gather(table_v, [sv + (r * N)]) * ex
            for r in range(RPT)]
    slot = jnp.bitwise_and(dv, SLOTS - 1)

    def _wcond(carry):
        it, rem = carry
        return jnp.logical_and(it < L, jnp.max(rem) > 0)

    def _wbody(carry):
        it, rem = carry
        remb = rem > 0
        plsc.store_scatter(slots_v, [slot], iota16, mask=remb)
        back = plsc.load_gather(slots_v, [slot])
        win = jnp.logical_and(back == iota16, remb)
        for r in range(RPT):
            plsc.addupdate_scatter(acc_v, [dv + (r * N)], vals[r], mask=win)
        plsc.addupdate_scatter(den_v, [dv], ex, mask=win)
        return (it + jnp.int32(1), jnp.where(win, 0, rem))

    lax.while_loop(_wcond, _wbody,
                   (jnp.int32(0), valid.astype(jnp.int32)))


def _one_gat(hs_hbm, als_hbm, ald_hbm, src_hbm, dst_hbm, num_hbm, den_hbm,
             table_v, acc_v, als_v, ald_v, den_v, slots_v,
             src_b, dst_b, ovf_b, sems, wid):
    base = wid * (RPT * N)
    cp_t = pltpu.async_copy(hs_hbm.at[pl.ds(base, RPT * N)], table_v,
                            sems.at[0, 0])
    cp_s = pltpu.async_copy(als_hbm, als_v, sems.at[0, 1])
    cp_d = pltpu.async_copy(ald_hbm, ald_v, sems.at[1, 0])

    zeros16 = jnp.zeros((L,), _f32)

    def _zacc(i, c):
        acc_v[pl.ds(i * L, L)] = zeros16
        return c
    lax.fori_loop(0, (RPT * N) // L, _zacc, 0)

    def _zden(i, c):
        den_v[pl.ds(i * L, L)] = zeros16
        return c
    lax.fori_loop(0, N // L, _zden, 0)
    cp_t.wait()
    cp_s.wait()
    cp_d.wait()

    iota16 = lax.iota(jnp.int32, L)
    ones_b = jnp.ones((L,), jnp.bool_)

    for b in range(2):
        pltpu.async_copy(src_hbm.at[pl.ds(b * CHUNK, CHUNK)], src_b[b],
                         sems.at[2, b])
        pltpu.async_copy(dst_hbm.at[pl.ds(b * CHUNK, CHUNK)], dst_b[b],
                         sems.at[3, b])

    def _pair(i, c):
        for b in range(2):
            ci = i * 2 + b
            pltpu.make_async_copy(src_hbm.at[pl.ds(0, CHUNK)], src_b[b],
                                  sems.at[2, b]).wait()
            pltpu.make_async_copy(dst_hbm.at[pl.ds(0, CHUNK)], dst_b[b],
                                  sems.at[3, b]).wait()
            cnt = jnp.int32(0)
            for j in range(CHUNK // L):
                sv = src_b[b][pl.ds(j * L, L)]
                dv = dst_b[b][pl.ds(j * L, L)]
                lose = _edge_group(sv, dv, table_v, acc_v, als_v, ald_v,
                                   den_v, slots_v, iota16, ones_b)
                pcnt = plsc.all_reduce_population_count(lose)[0]

                @pl.when(pcnt > 0)
                def _(lose=lose, cnt=cnt, pcnt=pcnt, j=j):
                    plsc.store_compressed(ovf_b.at[pl.ds(cnt, L)],
                                          iota16 + (j * L), mask=lose)
                cnt = cnt + pcnt

            @pl.when(cnt > 0)
            def _(cnt=cnt, b=b):
                def _ovf(g, c2):
                    pos = ovf_b[pl.ds(g * L, L)]
                    valid = (iota16 + g * L) < cnt
                    posc = jnp.where(valid, pos, 0)
                    sv = plsc.load_gather(src_b[b], [posc])
                    dv = plsc.load_gather(dst_b[b], [posc])
                    _resolve_group(sv, dv, table_v, acc_v, als_v, ald_v,
                                   den_v, slots_v, iota16, valid)
                    return c2
                lax.fori_loop(0, (cnt + (L - 1)) // L, _ovf, 0)

            nxt = ci + 2

            @pl.when(nxt < E // CHUNK)
            def _(nxt=nxt, b=b):
                off = nxt * CHUNK
                pltpu.async_copy(src_hbm.at[pl.ds(off, CHUNK)], src_b[b],
                                 sems.at[2, b])
                pltpu.async_copy(dst_hbm.at[pl.ds(off, CHUNK)], dst_b[b],
                                 sems.at[3, b])
        return c
    lax.fori_loop(0, E // (2 * CHUNK), _pair, 0)

    pltpu.sync_copy(acc_v, num_hbm.at[pl.ds(base, RPT * N)])

    @pl.when(wid == 0)
    def _():
        pltpu.sync_copy(den_v, den_hbm)


def _sc_layer_body(hsA_hbm, alsA_hbm, aldA_hbm, srcA_hbm, dstA_hbm,
                   hsB_hbm, alsB_hbm, aldB_hbm, srcB_hbm, dstB_hbm,
                   numA_hbm, denA_hbm, numB_hbm, denB_hbm,
                   table_v, acc_v, als_v, ald_v, den_v, slots_v,
                   src_b0, src_b1, dst_b0, dst_b1, ovf_b, sems):
    wid = lax.axis_index("s") * NC + lax.axis_index("c")
    src_b = [src_b0, src_b1]
    dst_b = [dst_b0, dst_b1]
    _one_gat(hsA_hbm, alsA_hbm, aldA_hbm, srcA_hbm, dstA_hbm,
             numA_hbm, denA_hbm,
             table_v, acc_v, als_v, ald_v, den_v, slots_v,
             src_b, dst_b, ovf_b, sems, wid)
    _one_gat(hsB_hbm, alsB_hbm, aldB_hbm, srcB_hbm, dstB_hbm,
             numB_hbm, denB_hbm,
             table_v, acc_v, als_v, ald_v, den_v, slots_v,
             src_b, dst_b, ovf_b, sems, wid)


_sc_layer = pl.kernel(
    _sc_layer_body,
    out_type=(
        jax.ShapeDtypeStruct((D * N,), _f32),
        jax.ShapeDtypeStruct((N,), _f32),
        jax.ShapeDtypeStruct((D * N,), _f32),
        jax.ShapeDtypeStruct((N,), _f32),
    ),
    mesh=plsc.VectorSubcoreMesh(core_axis_name="c", subcore_axis_name="s"),
    compiler_params=pltpu.CompilerParams(needs_layout_passes=False),
    scratch_types=[
        pltpu.VMEM((RPT * N,), _f32),
        pltpu.VMEM((RPT * N,), _f32),
        pltpu.VMEM((N,), _f32),
        pltpu.VMEM((N,), _f32),
        pltpu.VMEM((N,), _f32),
        pltpu.VMEM((SLOTS,), jnp.int32),
        pltpu.VMEM((CHUNK,), jnp.int32),
        pltpu.VMEM((CHUNK,), jnp.int32),
        pltpu.VMEM((CHUNK,), jnp.int32),
        pltpu.VMEM((CHUNK,), jnp.int32),
        pltpu.VMEM((CHUNK,), jnp.int32),
        pltpu.SemaphoreType.DMA((4, 2)),
    ],
)


_DN_T = (((0,), (1,)), ((), ()))
_DN_00 = (((0,), (0,)), ((), ()))
_DN_01 = (((0,), (1,)), ((), ()))


def _elu(x):
    return jnp.where(x > 0, x, jnp.exp(x) - 1.0)


def _gat_prep(hsrcT, hdstT, w, a_s, a_d, hs_o, als_o, ald_o):
    hs = lax.dot_general(w, hsrcT, _DN_00, preferred_element_type=_f32)
    hs_o[...] = hs
    als_o[...] = lax.dot_general(hs, a_s, _DN_01, preferred_element_type=_f32)
    wd = lax.dot_general(w, a_d, (((1,), (1,)), ((), ())),
                         preferred_element_type=_f32)
    ald_o[...] = lax.dot_general(hdstT, wd, _DN_00,
                                 preferred_element_type=_f32)


def _tc_prep_body(xu, xi, wp, bp, wA, asA, adA, wB, asB, adB,
                  hsA_o, alsA_o, aldA_o, hsB_o, alsB_o, aldB_o):
    huT = lax.dot_general(wp[...], xu[...], _DN_T,
                          preferred_element_type=_f32) + bp[...]
    hiT = lax.dot_general(wp[...], xi[...], _DN_T,
                          preferred_element_type=_f32) + bp[...]
    _gat_prep(hiT, huT, wA[...], asA[...], adA[...], hsA_o, alsA_o, aldA_o)
    _gat_prep(huT, hiT, wB[...], asB[...], adB[...], hsB_o, alsB_o, aldB_o)


_tc_prep = pl.pallas_call(
    _tc_prep_body,
    out_shape=[
        jax.ShapeDtypeStruct((D, N), _f32),
        jax.ShapeDtypeStruct((N, 1), _f32),
        jax.ShapeDtypeStruct((N, 1), _f32),
        jax.ShapeDtypeStruct((D, N), _f32),
        jax.ShapeDtypeStruct((N, 1), _f32),
        jax.ShapeDtypeStruct((N, 1), _f32),
    ],
)


def _tc_layer_body(numA, denA, bA, numB, denB, bB, wA, asA, adA, wB, asB, adB,
                   hsA_o, alsA_o, aldA_o, hsB_o, alsB_o, aldB_o):
    huT = _elu(numA[...] / (denA[...] + 1e-16) + bA[...])
    hiT = _elu(numB[...] / (denB[...] + 1e-16) + bB[...])
    _gat_prep(hiT, huT, wA[...], asA[...], adA[...], hsA_o, alsA_o, aldA_o)
    _gat_prep(huT, hiT, wB[...], asB[...], adB[...], hsB_o, alsB_o, aldB_o)


_tc_layer = pl.pallas_call(
    _tc_layer_body,
    out_shape=[
        jax.ShapeDtypeStruct((D, N), _f32),
        jax.ShapeDtypeStruct((N, 1), _f32),
        jax.ShapeDtypeStruct((N, 1), _f32),
        jax.ShapeDtypeStruct((D, N), _f32),
        jax.ShapeDtypeStruct((N, 1), _f32),
        jax.ShapeDtypeStruct((N, 1), _f32),
    ],
)


def _tc_final_body(numA, denA, bA, numB, denB, bB, wo, bo, outu_o, hiT_o):
    hu2T = _elu(numA[...] / (denA[...] + 1e-16) + bA[...])
    hiT_o[...] = _elu(numB[...] / (denB[...] + 1e-16) + bB[...])
    outu_o[...] = lax.dot_general(hu2T, wo[...], _DN_00,
                                  preferred_element_type=_f32) + bo[...]


_tc_final = pl.pallas_call(
    _tc_final_body,
    out_shape=[
        jax.ShapeDtypeStruct((N, OUT), _f32),
        jax.ShapeDtypeStruct((D, N), _f32),
    ],
)


def kernel(x_user, x_item, edge_index_ui, edge_index_iu, Wp, bp,
           W_ui0, as_ui0, ad_ui0, b_ui0, W_iu0, as_iu0, ad_iu0, b_iu0,
           W_ui1, as_ui1, ad_ui1, b_ui1, W_iu1, as_iu1, ad_iu1, b_iu1,
           Wo, bo):
    srcA = edge_index_iu[0].astype(jnp.int32)
    dstA = edge_index_iu[1].astype(jnp.int32)
    srcB = edge_index_ui[0].astype(jnp.int32)
    dstB = edge_index_ui[1].astype(jnp.int32)

    hsA, alsA, aldA, hsB, alsB, aldB = _tc_prep(
        x_user, x_item, Wp, bp.reshape(D, 1),
        W_iu0, as_iu0, ad_iu0, W_ui0, as_ui0, ad_ui0)

    numA, denA, numB, denB = _sc_layer(
        hsA.reshape(-1), alsA.reshape(-1), aldA.reshape(-1), srcA, dstA,
        hsB.reshape(-1), alsB.reshape(-1), aldB.reshape(-1), srcB, dstB)

    hsA1, alsA1, aldA1, hsB1, alsB1, aldB1 = _tc_layer(
        numA.reshape(D, N), denA.reshape(1, N), b_iu0.reshape(D, 1),
        numB.reshape(D, N), denB.reshape(1, N), b_ui0.reshape(D, 1),
        W_iu1, as_iu1, ad_iu1, W_ui1, as_ui1, ad_ui1)

    numA1, denA1, numB1, denB1 = _sc_layer(
        hsA1.reshape(-1), alsA1.reshape(-1), aldA1.reshape(-1), srcA, dstA,
        hsB1.reshape(-1), alsB1.reshape(-1), aldB1.reshape(-1), srcB, dstB)

    out_user, hi2T = _tc_final(
        numA1.reshape(D, N), denA1.reshape(1, N), b_iu1.reshape(D, 1),
        numB1.reshape(D, N), denB1.reshape(1, N), b_ui1.reshape(D, 1),
        Wo, bo.reshape(1, OUT))

    return (out_user, jnp.transpose(hi2T))
